# baseline (speedup 1.0000x reference)
import functools

import jax
import jax.numpy as jnp
from jax import lax
from jax.experimental import pallas as pl
from jax.experimental.pallas import tpu as pltpu
from jax.experimental.pallas import tpu_sc as plsc

N_V = 10000
N_E = 20000
NNZ = 320000
D = 128

NC = 2
NS = 16
L = 16
H = D // NC
CHUNK = 128
CW = 16


def _seg_sum_sc(num_seg, RK, U):
  IK = U
  n_chunks = NNZ // CHUNK
  base_iters = n_chunks // NS
  extra = n_chunks - base_iters * NS
  assert base_iters % U == 0 and U % RK == 0 and IK == U

  G8 = 8
  ngrp = num_seg // G8
  gb = ngrp // NS
  gx = ngrp - gb * NS
  main_rows = gb * G8
  ZB = 48
  assert main_rows % ZB == 0

  def _m8(x):
    return pl.multiple_of(x, G8)

  def body(table2, gids2, sids2, sums_out, cnts_out,
           gidr, sidr, rows, ones, zcnt, acc, cacc, isems, gsems):
    c = lax.axis_index("c")
    s = lax.axis_index("s")

    zv = jnp.zeros((L,), jnp.float32)
    ov = jnp.full((L,), 1.0, jnp.float32)
    for i in range(CHUNK):
      ones[i, :] = ov
    for i in range(ZB):
      zcnt[i, :] = zv
    for i in range(CHUNK):
      for k in range(H // L):
        rows[0][i, pl.ds(k * L, L)] = zv

    grow = _m8((s * gb + jnp.minimum(s, gx)) * G8)

    for z in range(main_rows // ZB):
      off = _m8(grow + z * ZB)
      pltpu.sync_copy(rows[0].at[pl.ds(0, ZB)], acc.at[pl.ds(off, ZB)])
      pltpu.sync_copy(zcnt.at[pl.ds(0, ZB)], cacc.at[pl.ds(off, ZB)])

    @pl.when(s < gx)
    def _():
      off = _m8(grow + main_rows)
      pltpu.sync_copy(rows[0].at[pl.ds(0, G8)], acc.at[pl.ds(off, G8)])
      pltpu.sync_copy(zcnt.at[pl.ds(0, G8)], cacc.at[pl.ds(off, G8)])

    plsc.subcore_barrier()

    chunk0 = s * base_iters

    def issue_ids(j, slot):
      pltpu.async_copy(gids2.at[pl.ds(chunk0 + j, 1)],
                       gidr.at[pl.ds(slot, 1)], isems[slot])
      pltpu.async_copy(sids2.at[pl.ds(chunk0 + j, 1)],
                       sidr.at[pl.ds(slot, 1)], isems[slot])

    def wait_ids(slot):
      pltpu.make_async_copy(gids2.at[pl.ds(0, 1)], gidr.at[pl.ds(slot, 1)],
                            isems[slot]).wait()
      pltpu.make_async_copy(sids2.at[pl.ds(0, 1)], sidr.at[pl.ds(slot, 1)],
                            isems[slot]).wait()

    def transform(idx_ref, slot):
      for k in range(CHUNK // L):
        v = idx_ref[slot, pl.ds(k * L, L)]
        idx_ref[slot, pl.ds(k * L, L)] = v * 2 + c

    def issue_gather(slot, ru):
      transform(gidr, slot)
      pltpu.async_copy(table2.at[gidr.at[slot]], rows[ru], gsems[ru])

    def drain_gather(ru):
      pltpu.make_async_copy(table2.at[pl.ds(0, CHUNK)], rows[ru],
                            gsems[ru]).wait()

    def scatter(slot, ru, count_core):
      pltpu.sync_copy(rows[ru], acc.at[sidr.at[slot]], add=True)

      @pl.when(count_core == c)
      def _():
        pltpu.sync_copy(ones, cacc.at[sidr.at[slot]], add=True)

    for u in range(IK):
      issue_ids(u, u)
    for u in range(RK):
      wait_ids(u)
      issue_gather(u, u)

    n_it = base_iters // U

    def it(i, carry):
      for u in range(U):
        j = i * U + u
        drain_gather(u % RK)
        scatter(u, u % RK, u % 2)

        @pl.when(i < n_it - 1)
        def _():
          issue_ids(j + IK, u)

        if u + RK < U:
          wait_ids(u + RK)
          issue_gather(u + RK, (u + RK) % RK)
        else:
          @pl.when(i < n_it - 1)
          def _():
            wait_ids((u + RK) % IK)
            issue_gather((u + RK) % IK, (u + RK) % RK)
      return carry

    lax.fori_loop(0, n_it, it, 0)

    @pl.when(s < extra)
    def _():
      ch = NS * base_iters + s
      pltpu.sync_copy(gids2.at[pl.ds(ch, 1)], gidr.at[pl.ds(0, 1)])
      pltpu.sync_copy(sids2.at[pl.ds(ch, 1)], sidr.at[pl.ds(0, 1)])
      transform(gidr, 0)
      pltpu.async_copy(table2.at[gidr.at[0]], rows[0], gsems[0]).wait()
      pltpu.sync_copy(rows[0], acc.at[sidr.at[0]], add=True)

      @pl.when((ch % 2) == c)
      def _():
        pltpu.sync_copy(ones, cacc.at[sidr.at[0]], add=True)

    plsc.subcore_barrier()

    pltpu.sync_copy(acc.at[pl.ds(grow, main_rows)],
                    sums_out.at[pl.ds(_m8(c * num_seg + grow), main_rows)])
    pltpu.sync_copy(cacc.at[pl.ds(grow, main_rows)],
                    cnts_out.at[pl.ds(_m8(c * num_seg + grow), main_rows)])

    @pl.when(s < gx)
    def _():
      off = _m8(grow + main_rows)
      pltpu.sync_copy(acc.at[pl.ds(off, G8)],
                      sums_out.at[pl.ds(_m8(c * num_seg + off), G8)])
      pltpu.sync_copy(cacc.at[pl.ds(off, G8)],
                      cnts_out.at[pl.ds(_m8(c * num_seg + off), G8)])

  mesh = plsc.VectorSubcoreMesh(core_axis_name="c", subcore_axis_name="s",
                                num_cores=NC, num_subcores=NS)
  return pl.kernel(
      body,
      out_type=[jax.ShapeDtypeStruct((NC * num_seg, H), jnp.float32),
                jax.ShapeDtypeStruct((NC * num_seg, CW), jnp.float32)],
      mesh=mesh,
      scratch_types=[
          pltpu.VMEM((IK, CHUNK), jnp.int32),
          pltpu.VMEM((IK, CHUNK), jnp.int32),
          [pltpu.VMEM((CHUNK, H), jnp.float32) for _ in range(RK)],
          pltpu.VMEM((CHUNK, CW), jnp.float32),
          pltpu.VMEM((ZB, CW), jnp.float32),
          pltpu.VMEM_SHARED((num_seg, H), jnp.float32),
          pltpu.VMEM_SHARED((num_seg, CW), jnp.float32),
          [pltpu.SemaphoreType.DMA for _ in range(IK)],
          [pltpu.SemaphoreType.DMA for _ in range(RK)],
      ],
      compiler_params=pltpu.CompilerParams(use_tc_tiling_on_sc=False),
  )


def _mm(a, b):
  return lax.dot_general(a, b, (((1,), (0,)), ((), ())),
                         preferred_element_type=jnp.float32,
                         precision=lax.Precision.HIGHEST)


def _ln_relu(x, g, b):
  mu = jnp.mean(x, axis=-1, keepdims=True)
  var = jnp.mean((x - mu) ** 2, axis=-1, keepdims=True)
  return jnp.maximum((x - mu) * lax.rsqrt(var + 1e-5) * g + b, 0.0)


_BR_E = 4000
_BR_V = 5000


def _edge_body(s01_ref0, s01_ref1, cnt_ref0, cnt_ref1, y_ref, wa_ref, wb_ref,
               c1_ref, c2_ref, ge_ref, beln_ref, ye_ref, yo_ref):
  cnt = cnt_ref0[:, 0:1] + cnt_ref1[:, 0:1]
  mx = jnp.concatenate([s01_ref0[...], s01_ref1[...]], axis=1)
  mx = mx / jnp.maximum(cnt, 1.0)
  agg = jnp.where(cnt > 0.0, _mm(mx, wb_ref[...]) + c2_ref[...], 0.0)
  ye = _mm(y_ref[...], wa_ref[...]) + agg + c1_ref[...]
  ye_ref[...] = ye
  yo_ref[...] = _ln_relu(ye, ge_ref[...], beln_ref[...])


def _vert_body(s01_ref0, s01_ref1, cnt_ref0, cnt_ref1, x_ref, wvmt_ref,
               wvmb_ref, bvm_ref, gv_ref, bvln_ref, xo_ref):
  cnt = cnt_ref0[:, 0:1] + cnt_ref1[:, 0:1]
  agg = jnp.concatenate([s01_ref0[...], s01_ref1[...]], axis=1)
  agg = agg / jnp.maximum(cnt, 1.0)
  xc = (_mm(x_ref[...], wvmt_ref[...]) + _mm(agg, wvmb_ref[...])
        + bvm_ref[...])
  xo_ref[...] = _ln_relu(xc, gv_ref[...], bvln_ref[...])


def _row_specs(num_seg, br):
  nb = num_seg // br
  s0 = pl.BlockSpec((br, H), lambda i: (i, 0))
  s1 = pl.BlockSpec((br, H), lambda i, nb=nb: (nb + i, 0))
  c0 = pl.BlockSpec((br, CW), lambda i: (i, 0))
  c1 = pl.BlockSpec((br, CW), lambda i, nb=nb: (nb + i, 0))
  row = pl.BlockSpec((br, D), lambda i: (i, 0))
  w = pl.BlockSpec((D, D), lambda i: (0, 0))
  b = pl.BlockSpec((1, D), lambda i: (0, 0))
  return nb, s0, s1, c0, c1, row, w, b


def _edge_tc(sums, cnts, Y, Wv, bv, Wem, bem, We, be, ge, beln):
  nb, s0, s1, c0, c1, row, w, b = _row_specs(N_E, _BR_E)
  wbe = _mm(Wem[D:], We)
  wa = _mm(Wem[:D], We)
  wb = _mm(Wv, wbe)
  cb2 = _mm(bv.reshape(1, D), wbe)
  cb1 = _mm(bem.reshape(1, D), We) + be.reshape(1, D)
  return pl.pallas_call(
      _edge_body,
      grid=(nb,),
      in_specs=[s0, s1, c0, c1, row, w, w, b, b, b, b],
      out_specs=[row, row],
      out_shape=[jax.ShapeDtypeStruct((N_E, D), jnp.float32),
                 jax.ShapeDtypeStruct((N_E, D), jnp.float32)],
      compiler_params=pltpu.CompilerParams(
          dimension_semantics=("parallel",)),
  )(sums, sums, cnts, cnts, Y, wa, wb, cb1, cb2, ge.reshape(1, D),
    beln.reshape(1, D))


def _vert_tc(sums, cnts, X, Wvm, bvm, gv, bvln):
  nb, s0, s1, c0, c1, row, w, b = _row_specs(N_V, _BR_V)
  wvm_t = pl.BlockSpec((D, D), lambda i: (0, 0))
  wvm_b = pl.BlockSpec((D, D), lambda i: (1, 0))
  return pl.pallas_call(
      _vert_body,
      grid=(nb,),
      in_specs=[s0, s1, c0, c1, row, wvm_t, wvm_b, b, b, b],
      out_specs=row,
      out_shape=jax.ShapeDtypeStruct((N_V, D), jnp.float32),
      compiler_params=pltpu.CompilerParams(
          dimension_semantics=("parallel",)),
  )(sums, sums, cnts, cnts, X, Wvm, Wvm, bvm.reshape(1, D),
    gv.reshape(1, D), bvln.reshape(1, D))


_seg_sum_cached = functools.cache(_seg_sum_sc)


def kernel(X, Y, vertex_ids, edge_ids, Wv, bv, We, be, Wem, bem, Wvm, bvm,
           gv, bvln, ge, beln):
  vid2 = vertex_ids.reshape(NNZ // CHUNK, CHUNK)
  eid2 = edge_ids.reshape(NNZ // CHUNK, CHUNK)
  esums, ecnt = _seg_sum_cached(N_E, 3, 6)(X.reshape(N_V * NC, H), vid2, eid2)
  Ye, Yo = _edge_tc(esums, ecnt, Y, Wv, bv, Wem, bem, We, be, ge, beln)
  vsums, vcnt = _seg_sum_cached(N_V, 6, 12)(Ye.reshape(N_E * NC, H), eid2,
                                            vid2)
  Xo = _vert_tc(vsums, vcnt, X, Wvm, bvm, gv, bvln)
  return (Xo, Yo)


# speedup vs baseline: 1.0231x; 1.0231x over previous
import functools

import jax
import jax.numpy as jnp
from jax import lax
from jax.experimental import pallas as pl
from jax.experimental.pallas import tpu as pltpu
from jax.experimental.pallas import tpu_sc as plsc

N_V = 10000
N_E = 20000
NNZ = 320000
D = 128

NC = 2
NS = 16
L = 16
H = D // NC
CHUNK = 128
CW = 16


def _seg_sum_sc(num_seg, RK, U):
  IK = U
  n_chunks = NNZ // CHUNK
  base_iters = n_chunks // NS
  extra = n_chunks - base_iters * NS
  assert base_iters % U == 0 and U % RK == 0 and IK == U

  G8 = 8
  ngrp = num_seg // G8
  gb = ngrp // NS
  gx = ngrp - gb * NS
  main_rows = gb * G8
  ZB = 48
  assert main_rows % ZB == 0

  def _m8(x):
    return pl.multiple_of(x, G8)

  def body(table2, gids2, sids2, sums_out, cnts_out,
           gidr, sidr, rows, ones, zcnt, acc, cacc, isems, gsems):
    c = lax.axis_index("c")
    s = lax.axis_index("s")

    zv = jnp.zeros((L,), jnp.float32)
    ov = jnp.full((L,), 1.0, jnp.float32)
    for i in range(CHUNK):
      ones[i, :] = ov
    for i in range(ZB):
      zcnt[i, :] = zv
    for i in range(CHUNK):
      for k in range(H // L):
        rows[0][i, pl.ds(k * L, L)] = zv

    grow = _m8((s * gb + jnp.minimum(s, gx)) * G8)

    for z in range(main_rows // ZB):
      off = _m8(grow + z * ZB)
      pltpu.sync_copy(rows[0].at[pl.ds(0, ZB)], acc.at[pl.ds(off, ZB)])
      pltpu.sync_copy(zcnt.at[pl.ds(0, ZB)], cacc.at[pl.ds(off, ZB)])

    @pl.when(s < gx)
    def _():
      off = _m8(grow + main_rows)
      pltpu.sync_copy(rows[0].at[pl.ds(0, G8)], acc.at[pl.ds(off, G8)])
      pltpu.sync_copy(zcnt.at[pl.ds(0, G8)], cacc.at[pl.ds(off, G8)])

    plsc.subcore_barrier()

    chunk0 = s * base_iters

    def issue_ids(j, slot):
      pltpu.async_copy(gids2.at[pl.ds(chunk0 + j, 1)],
                       gidr.at[pl.ds(slot, 1)], isems[slot])
      pltpu.async_copy(sids2.at[pl.ds(chunk0 + j, 1)],
                       sidr.at[pl.ds(slot, 1)], isems[slot])

    def wait_ids(slot):
      pltpu.make_async_copy(gids2.at[pl.ds(0, 1)], gidr.at[pl.ds(slot, 1)],
                            isems[slot]).wait()
      pltpu.make_async_copy(sids2.at[pl.ds(0, 1)], sidr.at[pl.ds(slot, 1)],
                            isems[slot]).wait()

    def transform(idx_ref, slot):
      for k in range(CHUNK // L):
        v = idx_ref[slot, pl.ds(k * L, L)]
        idx_ref[slot, pl.ds(k * L, L)] = v * 2 + c

    def issue_gather(slot, ru):
      transform(gidr, slot)
      pltpu.async_copy(table2.at[gidr.at[slot]], rows[ru], gsems[ru])

    def drain_gather(ru):
      pltpu.make_async_copy(table2.at[pl.ds(0, CHUNK)], rows[ru],
                            gsems[ru]).wait()

    def scatter(slot, ru, count_core):
      pltpu.sync_copy(rows[ru], acc.at[sidr.at[slot]], add=True)

      @pl.when(count_core == c)
      def _():
        pltpu.sync_copy(ones, cacc.at[sidr.at[slot]], add=True)

    for u in range(IK):
      issue_ids(u, u)
    for u in range(RK):
      wait_ids(u)
      issue_gather(u, u)

    n_it = base_iters // U

    def it(i, carry):
      for u in range(U):
        j = i * U + u
        drain_gather(u % RK)
        scatter(u, u % RK, u % 2)

        @pl.when(i < n_it - 1)
        def _():
          issue_ids(j + IK, u)

        if u + RK < U:
          wait_ids(u + RK)
          issue_gather(u + RK, (u + RK) % RK)
        else:
          @pl.when(i < n_it - 1)
          def _():
            wait_ids((u + RK) % IK)
            issue_gather((u + RK) % IK, (u + RK) % RK)
      return carry

    lax.fori_loop(0, n_it, it, 0)

    @pl.when(s < extra)
    def _():
      ch = NS * base_iters + s
      pltpu.sync_copy(gids2.at[pl.ds(ch, 1)], gidr.at[pl.ds(0, 1)])
      pltpu.sync_copy(sids2.at[pl.ds(ch, 1)], sidr.at[pl.ds(0, 1)])
      transform(gidr, 0)
      pltpu.async_copy(table2.at[gidr.at[0]], rows[0], gsems[0]).wait()
      pltpu.sync_copy(rows[0], acc.at[sidr.at[0]], add=True)

      @pl.when((ch % 2) == c)
      def _():
        pltpu.sync_copy(ones, cacc.at[sidr.at[0]], add=True)

    plsc.subcore_barrier()

    pltpu.sync_copy(acc.at[pl.ds(grow, main_rows)],
                    sums_out.at[pl.ds(_m8(c * num_seg + grow), main_rows)])
    pltpu.sync_copy(cacc.at[pl.ds(grow, main_rows)],
                    cnts_out.at[pl.ds(_m8(c * num_seg + grow), main_rows)])

    @pl.when(s < gx)
    def _():
      off = _m8(grow + main_rows)
      pltpu.sync_copy(acc.at[pl.ds(off, G8)],
                      sums_out.at[pl.ds(_m8(c * num_seg + off), G8)])
      pltpu.sync_copy(cacc.at[pl.ds(off, G8)],
                      cnts_out.at[pl.ds(_m8(c * num_seg + off), G8)])

  mesh = plsc.VectorSubcoreMesh(core_axis_name="c", subcore_axis_name="s",
                                num_cores=NC, num_subcores=NS)
  return pl.kernel(
      body,
      out_type=[jax.ShapeDtypeStruct((NC * num_seg, H), jnp.float32),
                jax.ShapeDtypeStruct((NC * num_seg, CW), jnp.float32)],
      mesh=mesh,
      scratch_types=[
          pltpu.VMEM((IK, CHUNK), jnp.int32),
          pltpu.VMEM((IK, CHUNK), jnp.int32),
          [pltpu.VMEM((CHUNK, H), jnp.float32) for _ in range(RK)],
          pltpu.VMEM((CHUNK, CW), jnp.float32),
          pltpu.VMEM((ZB, CW), jnp.float32),
          pltpu.VMEM_SHARED((num_seg, H), jnp.float32),
          pltpu.VMEM_SHARED((num_seg, CW), jnp.float32),
          [pltpu.SemaphoreType.DMA for _ in range(IK)],
          [pltpu.SemaphoreType.DMA for _ in range(RK)],
      ],
      compiler_params=pltpu.CompilerParams(use_tc_tiling_on_sc=False),
  )


def _mm(a, b):
  return lax.dot_general(a, b, (((1,), (0,)), ((), ())),
                         preferred_element_type=jnp.float32,
                         precision=lax.Precision.HIGHEST)


def _ln_relu(x, g, b):
  mu = jnp.mean(x, axis=-1, keepdims=True)
  var = jnp.mean((x - mu) ** 2, axis=-1, keepdims=True)
  return jnp.maximum((x - mu) * lax.rsqrt(var + 1e-5) * g + b, 0.0)


_BR_E = 2000
_BR_V = 2000


def _edge_body(s01_ref0, s01_ref1, cnt_ref0, cnt_ref1, y_ref, wa_ref, wb_ref,
               c1_ref, c2_ref, ge_ref, beln_ref, ye_ref, yo_ref):
  cnt = cnt_ref0[:, 0:1] + cnt_ref1[:, 0:1]
  mx = jnp.concatenate([s01_ref0[...], s01_ref1[...]], axis=1)
  mx = mx / jnp.maximum(cnt, 1.0)
  agg = jnp.where(cnt > 0.0, _mm(mx, wb_ref[...]) + c2_ref[...], 0.0)
  ye = _mm(y_ref[...], wa_ref[...]) + agg + c1_ref[...]
  ye_ref[...] = ye
  yo_ref[...] = _ln_relu(ye, ge_ref[...], beln_ref[...])


def _vert_body(s01_ref0, s01_ref1, cnt_ref0, cnt_ref1, x_ref, wvmt_ref,
               wvmb_ref, bvm_ref, gv_ref, bvln_ref, xo_ref):
  cnt = cnt_ref0[:, 0:1] + cnt_ref1[:, 0:1]
  agg = jnp.concatenate([s01_ref0[...], s01_ref1[...]], axis=1)
  agg = agg / jnp.maximum(cnt, 1.0)
  xc = (_mm(x_ref[...], wvmt_ref[...]) + _mm(agg, wvmb_ref[...])
        + bvm_ref[...])
  xo_ref[...] = _ln_relu(xc, gv_ref[...], bvln_ref[...])


def _row_specs(num_seg, br):
  nb = num_seg // br
  s0 = pl.BlockSpec((br, H), lambda i: (i, 0))
  s1 = pl.BlockSpec((br, H), lambda i, nb=nb: (nb + i, 0))
  c0 = pl.BlockSpec((br, CW), lambda i: (i, 0))
  c1 = pl.BlockSpec((br, CW), lambda i, nb=nb: (nb + i, 0))
  row = pl.BlockSpec((br, D), lambda i: (i, 0))
  w = pl.BlockSpec((D, D), lambda i: (0, 0))
  b = pl.BlockSpec((1, D), lambda i: (0, 0))
  return nb, s0, s1, c0, c1, row, w, b


def _edge_tc(sums, cnts, Y, Wv, bv, Wem, bem, We, be, ge, beln):
  nb, s0, s1, c0, c1, row, w, b = _row_specs(N_E, _BR_E)
  wbe = _mm(Wem[D:], We)
  wa = _mm(Wem[:D], We)
  wb = _mm(Wv, wbe)
  cb2 = _mm(bv.reshape(1, D), wbe)
  cb1 = _mm(bem.reshape(1, D), We) + be.reshape(1, D)
  return pl.pallas_call(
      _edge_body,
      grid=(nb,),
      in_specs=[s0, s1, c0, c1, row, w, w, b, b, b, b],
      out_specs=[row, row],
      out_shape=[jax.ShapeDtypeStruct((N_E, D), jnp.float32),
                 jax.ShapeDtypeStruct((N_E, D), jnp.float32)],
      compiler_params=pltpu.CompilerParams(
          dimension_semantics=("parallel",)),
  )(sums, sums, cnts, cnts, Y, wa, wb, cb1, cb2, ge.reshape(1, D),
    beln.reshape(1, D))


def _vert_tc(sums, cnts, X, Wvm, bvm, gv, bvln):
  nb, s0, s1, c0, c1, row, w, b = _row_specs(N_V, _BR_V)
  wvm_t = pl.BlockSpec((D, D), lambda i: (0, 0))
  wvm_b = pl.BlockSpec((D, D), lambda i: (1, 0))
  return pl.pallas_call(
      _vert_body,
      grid=(nb,),
      in_specs=[s0, s1, c0, c1, row, wvm_t, wvm_b, b, b, b],
      out_specs=row,
      out_shape=jax.ShapeDtypeStruct((N_V, D), jnp.float32),
      compiler_params=pltpu.CompilerParams(
          dimension_semantics=("parallel",)),
  )(sums, sums, cnts, cnts, X, Wvm, Wvm, bvm.reshape(1, D),
    gv.reshape(1, D), bvln.reshape(1, D))


_seg_sum_cached = functools.cache(_seg_sum_sc)


def kernel(X, Y, vertex_ids, edge_ids, Wv, bv, We, be, Wem, bem, Wvm, bvm,
           gv, bvln, ge, beln):
  vid2 = vertex_ids.reshape(NNZ // CHUNK, CHUNK)
  eid2 = edge_ids.reshape(NNZ // CHUNK, CHUNK)
  esums, ecnt = _seg_sum_cached(N_E, 3, 6)(X.reshape(N_V * NC, H), vid2, eid2)
  Ye, Yo = _edge_tc(esums, ecnt, Y, Wv, bv, Wem, bem, We, be, ge, beln)
  vsums, vcnt = _seg_sum_cached(N_V, 6, 12)(Ye.reshape(N_E * NC, H), eid2,
                                            vid2)
  Xo = _vert_tc(vsums, vcnt, X, Wvm, bvm, gv, bvln)
  return (Xo, Yo)


# speedup vs baseline: 1.0355x; 1.0121x over previous
import functools

import jax
import jax.numpy as jnp
from jax import lax
from jax.experimental import pallas as pl
from jax.experimental.pallas import tpu as pltpu
from jax.experimental.pallas import tpu_sc as plsc

N_V = 10000
N_E = 20000
NNZ = 320000
D = 128

NC = 2
NS = 16
L = 16
H = D // NC
CHUNK = 128
CW = 16


def _seg_sum_sc(num_seg, RK, U):
  IK = U
  n_chunks = NNZ // CHUNK
  base_iters = n_chunks // NS
  extra = n_chunks - base_iters * NS
  assert base_iters % U == 0 and U % RK == 0 and IK == U

  G8 = 8
  ngrp = num_seg // G8
  gb = ngrp // NS
  gx = ngrp - gb * NS
  main_rows = gb * G8
  ZB = 48
  assert main_rows % ZB == 0

  def _m8(x):
    return pl.multiple_of(x, G8)

  def body(table2, gids2, sids2, sums_out, cnts_out,
           gidr, sidr, rows, ones, zcnt, acc, cacc, isems, gsems):
    c = lax.axis_index("c")
    s = lax.axis_index("s")

    zv = jnp.zeros((L,), jnp.float32)
    ov = jnp.full((L,), 1.0, jnp.float32)
    for i in range(CHUNK):
      ones[i, :] = ov
    for i in range(ZB):
      zcnt[i, :] = zv
    for i in range(CHUNK):
      for k in range(H // L):
        rows[0][i, pl.ds(k * L, L)] = zv

    grow = _m8((s * gb + jnp.minimum(s, gx)) * G8)

    for z in range(main_rows // ZB):
      off = _m8(grow + z * ZB)
      pltpu.sync_copy(rows[0].at[pl.ds(0, ZB)], acc.at[pl.ds(off, ZB)])
      pltpu.sync_copy(zcnt.at[pl.ds(0, ZB)], cacc.at[pl.ds(off, ZB)])

    @pl.when(s < gx)
    def _():
      off = _m8(grow + main_rows)
      pltpu.sync_copy(rows[0].at[pl.ds(0, G8)], acc.at[pl.ds(off, G8)])
      pltpu.sync_copy(zcnt.at[pl.ds(0, G8)], cacc.at[pl.ds(off, G8)])

    plsc.subcore_barrier()

    chunk0 = s * base_iters

    def issue_ids(j, slot):
      pltpu.async_copy(gids2.at[pl.ds(chunk0 + j, 1)],
                       gidr.at[pl.ds(slot, 1)], isems[slot])
      pltpu.async_copy(sids2.at[pl.ds(chunk0 + j, 1)],
                       sidr.at[pl.ds(slot, 1)], isems[slot])

    def wait_ids(slot):
      pltpu.make_async_copy(gids2.at[pl.ds(0, 1)], gidr.at[pl.ds(slot, 1)],
                            isems[slot]).wait()
      pltpu.make_async_copy(sids2.at[pl.ds(0, 1)], sidr.at[pl.ds(slot, 1)],
                            isems[slot]).wait()

    def transform(idx_ref, slot):
      for k in range(CHUNK // L):
        v = idx_ref[slot, pl.ds(k * L, L)]
        idx_ref[slot, pl.ds(k * L, L)] = v * 2 + c

    def issue_gather(slot, ru):
      transform(gidr, slot)
      pltpu.async_copy(table2.at[gidr.at[slot]], rows[ru], gsems[ru])

    def drain_gather(ru):
      pltpu.make_async_copy(table2.at[pl.ds(0, CHUNK)], rows[ru],
                            gsems[ru]).wait()

    def scatter(slot, ru, count_core):
      pltpu.sync_copy(rows[ru], acc.at[sidr.at[slot]], add=True)

      @pl.when(count_core == c)
      def _():
        pltpu.sync_copy(ones, cacc.at[sidr.at[slot]], add=True)

    for u in range(IK):
      issue_ids(u, u)
    for u in range(RK):
      wait_ids(u)
      issue_gather(u, u)

    n_it = base_iters // U

    def it(i, carry):
      for u in range(U):
        j = i * U + u
        drain_gather(u % RK)
        scatter(u, u % RK, u % 2)

        @pl.when(i < n_it - 1)
        def _():
          issue_ids(j + IK, u)

        if u + RK < U:
          wait_ids(u + RK)
          issue_gather(u + RK, (u + RK) % RK)
        else:
          @pl.when(i < n_it - 1)
          def _():
            wait_ids((u + RK) % IK)
            issue_gather((u + RK) % IK, (u + RK) % RK)
      return carry

    lax.fori_loop(0, n_it, it, 0)

    @pl.when(s < extra)
    def _():
      ch = NS * base_iters + s
      pltpu.sync_copy(gids2.at[pl.ds(ch, 1)], gidr.at[pl.ds(0, 1)])
      pltpu.sync_copy(sids2.at[pl.ds(ch, 1)], sidr.at[pl.ds(0, 1)])
      transform(gidr, 0)
      pltpu.async_copy(table2.at[gidr.at[0]], rows[0], gsems[0]).wait()
      pltpu.sync_copy(rows[0], acc.at[sidr.at[0]], add=True)

      @pl.when((ch % 2) == c)
      def _():
        pltpu.sync_copy(ones, cacc.at[sidr.at[0]], add=True)

    plsc.subcore_barrier()

    pltpu.sync_copy(acc.at[pl.ds(grow, main_rows)],
                    sums_out.at[pl.ds(_m8(c * num_seg + grow), main_rows)])
    pltpu.sync_copy(cacc.at[pl.ds(grow, main_rows)],
                    cnts_out.at[pl.ds(_m8(c * num_seg + grow), main_rows)])

    @pl.when(s < gx)
    def _():
      off = _m8(grow + main_rows)
      pltpu.sync_copy(acc.at[pl.ds(off, G8)],
                      sums_out.at[pl.ds(_m8(c * num_seg + off), G8)])
      pltpu.sync_copy(cacc.at[pl.ds(off, G8)],
                      cnts_out.at[pl.ds(_m8(c * num_seg + off), G8)])

  mesh = plsc.VectorSubcoreMesh(core_axis_name="c", subcore_axis_name="s",
                                num_cores=NC, num_subcores=NS)
  return pl.kernel(
      body,
      out_type=[jax.ShapeDtypeStruct((NC * num_seg, H), jnp.float32),
                jax.ShapeDtypeStruct((NC * num_seg, CW), jnp.float32)],
      mesh=mesh,
      scratch_types=[
          pltpu.VMEM((IK, CHUNK), jnp.int32),
          pltpu.VMEM((IK, CHUNK), jnp.int32),
          [pltpu.VMEM((CHUNK, H), jnp.float32) for _ in range(RK)],
          pltpu.VMEM((CHUNK, CW), jnp.float32),
          pltpu.VMEM((ZB, CW), jnp.float32),
          pltpu.VMEM_SHARED((num_seg, H), jnp.float32),
          pltpu.VMEM_SHARED((num_seg, CW), jnp.float32),
          [pltpu.SemaphoreType.DMA for _ in range(IK)],
          [pltpu.SemaphoreType.DMA for _ in range(RK)],
      ],
      compiler_params=pltpu.CompilerParams(use_tc_tiling_on_sc=False),
  )


def _mm(a, b):
  return lax.dot_general(a, b, (((1,), (0,)), ((), ())),
                         preferred_element_type=jnp.float32,
                         precision=lax.Precision.HIGHEST)


def _ln_relu(x, g, b):
  mu = jnp.mean(x, axis=-1, keepdims=True)
  var = jnp.mean((x - mu) ** 2, axis=-1, keepdims=True)
  return jnp.maximum((x - mu) * lax.rsqrt(var + 1e-5) * g + b, 0.0)


_BR_E = 2000
_BR_V = 2000


def _premm_body(a_ref, w_ref, o_ref):
  o_ref[...] = _mm(a_ref[...], w_ref[...])


def _premm(a, w, n, br):
  return pl.pallas_call(
      _premm_body,
      grid=(n // br,),
      in_specs=[pl.BlockSpec((br, D), lambda i: (i, 0)),
                pl.BlockSpec((D, D), lambda i: (0, 0))],
      out_specs=pl.BlockSpec((br, D), lambda i: (i, 0)),
      out_shape=jax.ShapeDtypeStruct((n, D), jnp.float32),
      compiler_params=pltpu.CompilerParams(
          dimension_semantics=("parallel",)),
  )(a, w)


def _edge_body(s01_ref0, s01_ref1, cnt_ref0, cnt_ref1, yp_ref, wb_ref,
               c1_ref, c2_ref, ge_ref, beln_ref, ye_ref, yo_ref):
  cnt = cnt_ref0[:, 0:1] + cnt_ref1[:, 0:1]
  mx = jnp.concatenate([s01_ref0[...], s01_ref1[...]], axis=1)
  mx = mx / jnp.maximum(cnt, 1.0)
  agg = jnp.where(cnt > 0.0, _mm(mx, wb_ref[...]) + c2_ref[...], 0.0)
  ye = yp_ref[...] + agg + c1_ref[...]
  ye_ref[...] = ye
  yo_ref[...] = _ln_relu(ye, ge_ref[...], beln_ref[...])


def _vert_body(s01_ref0, s01_ref1, cnt_ref0, cnt_ref1, xp_ref,
               wvmb_ref, bvm_ref, gv_ref, bvln_ref, xo_ref):
  cnt = cnt_ref0[:, 0:1] + cnt_ref1[:, 0:1]
  agg = jnp.concatenate([s01_ref0[...], s01_ref1[...]], axis=1)
  agg = agg / jnp.maximum(cnt, 1.0)
  xc = xp_ref[...] + _mm(agg, wvmb_ref[...]) + bvm_ref[...]
  xo_ref[...] = _ln_relu(xc, gv_ref[...], bvln_ref[...])


def _row_specs(num_seg, br):
  nb = num_seg // br
  s0 = pl.BlockSpec((br, H), lambda i: (i, 0))
  s1 = pl.BlockSpec((br, H), lambda i, nb=nb: (nb + i, 0))
  c0 = pl.BlockSpec((br, CW), lambda i: (i, 0))
  c1 = pl.BlockSpec((br, CW), lambda i, nb=nb: (nb + i, 0))
  row = pl.BlockSpec((br, D), lambda i: (i, 0))
  w = pl.BlockSpec((D, D), lambda i: (0, 0))
  b = pl.BlockSpec((1, D), lambda i: (0, 0))
  return nb, s0, s1, c0, c1, row, w, b


def _edge_tc(sums, cnts, Y, Wv, bv, Wem, bem, We, be, ge, beln):
  nb, s0, s1, c0, c1, row, w, b = _row_specs(N_E, _BR_E)
  wbe = _mm(Wem[D:], We)
  wa = _mm(Wem[:D], We)
  wb = _mm(Wv, wbe)
  cb2 = _mm(bv.reshape(1, D), wbe)
  cb1 = _mm(bem.reshape(1, D), We) + be.reshape(1, D)
  yp = _premm(Y, wa, N_E, _BR_E)
  return pl.pallas_call(
      _edge_body,
      grid=(nb,),
      in_specs=[s0, s1, c0, c1, row, w, b, b, b, b],
      out_specs=[row, row],
      out_shape=[jax.ShapeDtypeStruct((N_E, D), jnp.float32),
                 jax.ShapeDtypeStruct((N_E, D), jnp.float32)],
      compiler_params=pltpu.CompilerParams(
          dimension_semantics=("parallel",)),
  )(sums, sums, cnts, cnts, yp, wb, cb1, cb2, ge.reshape(1, D),
    beln.reshape(1, D))


def _vert_tc(sums, cnts, X, Wvm, bvm, gv, bvln):
  nb, s0, s1, c0, c1, row, w, b = _row_specs(N_V, _BR_V)
  xp = _premm(X, Wvm[:D], N_V, _BR_V)
  return pl.pallas_call(
      _vert_body,
      grid=(nb,),
      in_specs=[s0, s1, c0, c1, row, w, b, b, b],
      out_specs=row,
      out_shape=jax.ShapeDtypeStruct((N_V, D), jnp.float32),
      compiler_params=pltpu.CompilerParams(
          dimension_semantics=("parallel",)),
  )(sums, sums, cnts, cnts, xp, Wvm[D:], bvm.reshape(1, D),
    gv.reshape(1, D), bvln.reshape(1, D))


_seg_sum_cached = functools.cache(_seg_sum_sc)


def kernel(X, Y, vertex_ids, edge_ids, Wv, bv, We, be, Wem, bem, Wvm, bvm,
           gv, bvln, ge, beln):
  vid2 = vertex_ids.reshape(NNZ // CHUNK, CHUNK)
  eid2 = edge_ids.reshape(NNZ // CHUNK, CHUNK)
  esums, ecnt = _seg_sum_cached(N_E, 3, 6)(X.reshape(N_V * NC, H), vid2, eid2)
  Ye, Yo = _edge_tc(esums, ecnt, Y, Wv, bv, Wem, bem, We, be, ge, beln)
  vsums, vcnt = _seg_sum_cached(N_V, 6, 12)(Ye.reshape(N_E * NC, H), eid2,
                                            vid2)
  Xo = _vert_tc(vsums, vcnt, X, Wvm, bvm, gv, bvln)
  return (Xo, Yo)
